# baseline (speedup 1.0000x reference)
import functools

import jax
import jax.numpy as jnp
from jax import lax
from jax.experimental import pallas as pl
from jax.experimental.pallas import tpu as pltpu
from jax.experimental.pallas import tpu_sc as plsc

B, T = 8, 40
C, H, W = 3, 192, 192
F = C * H * W
NC, NS, L = 2, 16, 16
NW = NC * NS
NCH = 6
CH = F // NCH
UNROLL = 8
N_INNER = CH // (UNROLL * L)


def _chunk_sum(xbuf, ybuf):
    def step(i, acc):
        base = i * (UNROLL * L)
        for u in range(UNROLL):
            xv = xbuf[pl.ds(base + u * L, L)]
            yv = ybuf[pl.ds(base + u * L, L)]
            acc = acc + jnp.abs(xv - yv)
        return acc

    return lax.fori_loop(0, N_INNER, step, jnp.zeros((L,), jnp.float32))


@functools.partial(
    pl.kernel,
    out_type=jax.ShapeDtypeStruct((NW, L), jnp.float32),
    mesh=plsc.VectorSubcoreMesh(
        core_axis_name="c", subcore_axis_name="s", num_cores=NC, num_subcores=NS
    ),
    compiler_params=pltpu.CompilerParams(needs_layout_passes=False),
    scratch_types=[
        pltpu.VMEM((CH,), jnp.float32),
        pltpu.VMEM((CH,), jnp.float32),
        pltpu.VMEM((CH,), jnp.float32),
        pltpu.VMEM((CH,), jnp.float32),
        pltpu.VMEM((16,), jnp.int32),
        pltpu.VMEM((L,), jnp.float32),
        pltpu.SemaphoreType.DMA,
        pltpu.SemaphoreType.DMA,
        pltpu.SemaphoreType.DMA,
        pltpu.SemaphoreType.DMA,
    ],
)
def _sc_l1(x_hbm, y_hbm, vl_hbm, out_hbm,
           xb0, xb1, yb0, yb1, vlv, accv, sx0, sx1, sy0, sy1):
    cid = lax.axis_index("c")
    sid = lax.axis_index("s")
    wid = sid * NC + cid

    pltpu.sync_copy(vl_hbm, vlv)
    vl = vlv[...]
    cum = plsc.cumsum(vl)
    cumex = cum - vl
    nv = jnp.max(cum)
    iota = lax.iota(jnp.int32, 16)

    nf = (nv - wid + (NW - 1)) // NW
    nq = nf * NCH

    def chunk_off(q):
        k = q // NCH
        c = q - k * NCH
        j = wid + NW * k
        bb = jnp.sum((cum <= j).astype(jnp.int32))
        start = jnp.sum(jnp.where(iota == bb, cumex, 0))
        frame = bb * T + (j - start)
        return pl.multiple_of(frame * F + c * CH, CH)

    def start_q(q, xbuf, ybuf, sx, sy):
        off = chunk_off(q)
        pltpu.async_copy(x_hbm.at[pl.ds(off, CH)], xbuf, sx)
        pltpu.async_copy(y_hbm.at[pl.ds(off, CH)], ybuf, sy)

    def wait_q(xbuf, ybuf, sx, sy):
        pltpu.make_async_copy(x_hbm.at[pl.ds(0, CH)], xbuf, sx).wait()
        pltpu.make_async_copy(y_hbm.at[pl.ds(0, CH)], ybuf, sy).wait()

    @pl.when(nq > 0)
    def _():
        start_q(0, xb0, yb0, sx0, sy0)

    @pl.when(nq > 1)
    def _():
        start_q(1, xb1, yb1, sx1, sy1)

    def pair(g, acc):
        q0 = 2 * g
        q1 = q0 + 1
        wait_q(xb0, yb0, sx0, sy0)
        acc = acc + _chunk_sum(xb0, yb0)

        @pl.when(q0 + 2 < nq)
        def _():
            start_q(q0 + 2, xb0, yb0, sx0, sy0)

        @pl.when(q1 < nq)
        def _():
            wait_q(xb1, yb1, sx1, sy1)

        s1 = _chunk_sum(xb1, yb1)
        acc = acc + jnp.where(q1 < nq, s1, 0.0)

        @pl.when(q1 + 2 < nq)
        def _():
            start_q(q1 + 2, xb1, yb1, sx1, sy1)

        return acc

    acc = lax.fori_loop(0, (nq + 1) // 2, pair, jnp.zeros((L,), jnp.float32))
    accv[...] = acc
    pltpu.sync_copy(accv, out_hbm.at[wid])


def kernel(inputs, gt, valid_len):
    x = inputs.reshape(-1)
    y = gt.reshape(-1)
    vl32 = valid_len.astype(jnp.int32)
    vl_pad = jnp.zeros((16,), jnp.int32).at[:B].set(vl32)
    partials = _sc_l1(x, y, vl_pad)
    total = jnp.sum(partials)
    count = jnp.sum(valid_len).astype(inputs.dtype) * (C * H * W)
    return total / count


# speedup vs baseline: 3.9369x; 3.9369x over previous
import functools

import jax
import jax.numpy as jnp
from jax import lax
from jax.experimental import pallas as pl
from jax.experimental.pallas import tpu as pltpu
from jax.experimental.pallas import tpu_sc as plsc

B, T = 8, 40
C, H, W = 3, 192, 192
NC, NS, L = 2, 16, 16
NW = NC * NS
HH = H // 2
NCH = 2 * C
ROW_UNROLL = W // L


def _chunk_sum(xbuf, ybuf):
    def step(r, acc):
        for u in range(ROW_UNROLL):
            xv = xbuf[r, pl.ds(u * L, L)]
            yv = ybuf[r, pl.ds(u * L, L)]
            acc = acc + jnp.abs(xv - yv)
        return acc

    return lax.fori_loop(0, HH, step, jnp.zeros((L,), jnp.float32))


@functools.partial(
    pl.kernel,
    out_type=jax.ShapeDtypeStruct((NW, L), jnp.float32),
    mesh=plsc.VectorSubcoreMesh(
        core_axis_name="c", subcore_axis_name="s", num_cores=NC, num_subcores=NS
    ),
    compiler_params=pltpu.CompilerParams(needs_layout_passes=False),
    scratch_types=[
        pltpu.VMEM((HH, W), jnp.float32),
        pltpu.VMEM((HH, W), jnp.float32),
        pltpu.VMEM((HH, W), jnp.float32),
        pltpu.VMEM((HH, W), jnp.float32),
        pltpu.VMEM((16,), jnp.int32),
        pltpu.VMEM((L,), jnp.float32),
        pltpu.SemaphoreType.DMA,
        pltpu.SemaphoreType.DMA,
        pltpu.SemaphoreType.DMA,
        pltpu.SemaphoreType.DMA,
    ],
)
def _sc_l1(x_hbm, y_hbm, vl_hbm, out_hbm,
           xb0, xb1, yb0, yb1, vlv, accv, sx0, sx1, sy0, sy1):
    cid = lax.axis_index("c")
    sid = lax.axis_index("s")
    wid = sid * NC + cid

    pltpu.sync_copy(vl_hbm, vlv)
    vl = vlv[...]
    cum = plsc.cumsum(vl)
    cumex = cum - vl
    nv = jnp.max(cum)
    iota = lax.iota(jnp.int32, 16)

    nf = (nv - wid + (NW - 1)) // NW
    nq = nf * NCH

    def chunk_loc(q):
        k = q // NCH
        c = q - k * NCH
        j = wid + NW * k
        bb = jnp.sum((cum <= j).astype(jnp.int32))
        start = jnp.sum(jnp.where(iota == bb, cumex, 0))
        t = j - start
        ch = c // 2
        h0 = (c - 2 * ch) * HH
        return bb, t, ch, h0

    def start_q(q, xbuf, ybuf, sx, sy):
        bb, t, ch, h0 = chunk_loc(q)
        pltpu.async_copy(x_hbm.at[bb, t, ch, pl.ds(h0, HH), :], xbuf, sx)
        pltpu.async_copy(y_hbm.at[bb, t, ch, pl.ds(h0, HH), :], ybuf, sy)

    def wait_q(xbuf, ybuf, sx, sy):
        pltpu.make_async_copy(x_hbm.at[0, 0, 0, pl.ds(0, HH), :], xbuf, sx).wait()
        pltpu.make_async_copy(y_hbm.at[0, 0, 0, pl.ds(0, HH), :], ybuf, sy).wait()

    @pl.when(nq > 0)
    def _():
        start_q(0, xb0, yb0, sx0, sy0)

    @pl.when(nq > 1)
    def _():
        start_q(1, xb1, yb1, sx1, sy1)

    def pair(g, acc):
        q0 = 2 * g
        q1 = q0 + 1
        wait_q(xb0, yb0, sx0, sy0)
        acc = acc + _chunk_sum(xb0, yb0)

        @pl.when(q0 + 2 < nq)
        def _():
            start_q(q0 + 2, xb0, yb0, sx0, sy0)

        @pl.when(q1 < nq)
        def _():
            wait_q(xb1, yb1, sx1, sy1)

        s1 = _chunk_sum(xb1, yb1)
        acc = acc + jnp.where(q1 < nq, s1, 0.0)

        @pl.when(q1 + 2 < nq)
        def _():
            start_q(q1 + 2, xb1, yb1, sx1, sy1)

        return acc

    acc = lax.fori_loop(0, (nq + 1) // 2, pair, jnp.zeros((L,), jnp.float32))
    accv[...] = acc
    pltpu.sync_copy(accv, out_hbm.at[wid])


def kernel(inputs, gt, valid_len):
    vl32 = valid_len.astype(jnp.int32)
    vl_pad = jnp.zeros((16,), jnp.int32).at[:B].set(vl32)
    partials = _sc_l1(inputs, gt, vl_pad)
    total = jnp.sum(partials)
    count = jnp.sum(valid_len).astype(inputs.dtype) * (C * H * W)
    return total / count
